# baseline (speedup 1.0000x reference)
import functools

import jax
import jax.numpy as jnp
from jax import lax
from jax.experimental import pallas as pl
from jax.experimental.pallas import tpu as pltpu
from jax.experimental.pallas import tpu_sc as plsc

_NC = 2
_NS = 16
_NW = _NC * _NS
_CHUNK = 40


@functools.partial(jax.jit, static_argnames=("n", "d", "chunk"))
def _gather_rows(table, flat_idx, n, d, chunk):
    b_per_w = n // _NW
    n_chunks = b_per_w // chunk
    assert n_chunks % 2 == 0 and b_per_w % chunk == 0 and chunk % 8 == 0
    n_pairs = n_chunks // 2
    mesh = plsc.VectorSubcoreMesh(
        core_axis_name="c", subcore_axis_name="s",
        num_cores=_NC, num_subcores=_NS)

    @functools.partial(
        pl.kernel,
        out_type=jax.ShapeDtypeStruct((n, d), jnp.float32),
        mesh=mesh,
        scratch_types=[
            pltpu.VMEM((b_per_w,), jnp.int32),
            pltpu.VMEM((2, chunk, d), jnp.float32),
            pltpu.SemaphoreType.DMA((2,)),
            pltpu.SemaphoreType.DMA((2,)),
        ],
        compiler_params=pltpu.CompilerParams(use_tc_tiling_on_sc=False),
    )
    def run(table_hbm, idx_hbm, out_hbm, idx_v, rows_v, gsem, ssem):
        wid = lax.axis_index("s") * _NC + lax.axis_index("c")
        base = wid * b_per_w
        pltpu.sync_copy(idx_hbm.at[pl.ds(base, b_per_w)], idx_v)

        def gather(b, c):
            return pltpu.make_async_copy(
                table_hbm.at[idx_v.at[pl.ds(c * chunk, chunk)]],
                rows_v.at[b], gsem.at[b])

        def store(b, c):
            return pltpu.make_async_copy(
                rows_v.at[b], out_hbm.at[pl.ds(base + c * chunk, chunk)],
                ssem.at[b])

        gather(0, 0).start()
        gather(1, 1).start()

        @pl.loop(0, n_pairs)
        def _pair(g):
            c0 = 2 * g
            c1 = c0 + 1
            last = n_chunks - 1
            c2 = jnp.minimum(c0 + 2, last)
            c3 = jnp.minimum(c0 + 3, last)
            gather(0, c0).wait()
            store(0, c0).start()
            gather(1, c1).wait()
            store(1, c1).start()
            store(0, c0).wait()
            gather(0, c2).start()
            store(1, c1).wait()
            gather(1, c3).start()

        gather(0, n_chunks - 1).wait()
        gather(1, n_chunks - 1).wait()

    return run(table, flat_idx)


def kernel(idx, logits_table):
    b, s = idx.shape
    v, d = logits_table.shape
    del v
    flat = idx.reshape(b * s).astype(jnp.int32)
    out = _gather_rows(logits_table, flat, b * s, d, _CHUNK)
    return out.reshape(b, s, d)
